# baseline (speedup 1.0000x reference)
import functools
import numpy as np
import jax
import jax.numpy as jnp
from jax import lax
from jax.experimental import pallas as pl
from jax.experimental.pallas import tpu as pltpu
from jax.experimental.pallas import tpu_sc as plsc

N = 10000
E = 640000
MUL = 32
D = 4 * MUL
SILU_NORM = 1.679
_C = 1.0 / np.sqrt(2.0)
_ISQ3 = 1.0 / np.sqrt(3.0)
_ISQH = 1.0 / np.sqrt(32.0)

NC = 2
NS = 16
NW = NC * NS
CH = 80
NP = 10240
RPS = NP // NS
NCHUNK = 5
EC = E // NCHUNK

_S = np.zeros((96, 32), np.float32)
_R = np.zeros((32, 96), np.float32)
_T = np.zeros((3, 96), np.float32)
for _u in range(32):
    for _m in range(3):
        _S[3 * _u + _m, _u] = 1.0
        _R[_u, 3 * _u + _m] = 1.0
        _T[_m, 3 * _u + _m] = 1.0
_K = (np.arange(96)[:, None] % 3 == np.arange(96)[None, :] % 3)
_K = _K.astype(np.float32)

_BD = np.zeros((160, 224), np.float32)
_BD[0:96, 0:32] = _S
_BD[96:128, 32:128] = _R
_BD[128:160, 128:224] = _R


KG = 5
SUPG = KG * CH
KS = 2
SUPS = KS * CH


def _sc_gather(node_feat, src, ne):
    mesh = plsc.VectorSubcoreMesh(core_axis_name="c", subcore_axis_name="s")
    epw = ne // NW
    iters = epw // SUPG

    @functools.partial(
        pl.kernel, mesh=mesh,
        out_type=jax.ShapeDtypeStruct((ne, D), jnp.float32),
        scratch_types=[
            pltpu.VMEM((2, KG, CH), jnp.int32),
            pltpu.VMEM((2, SUPG, D), jnp.float32),
            pltpu.SemaphoreType.DMA,
            pltpu.SemaphoreType.DMA,
            pltpu.SemaphoreType.DMA,
            pltpu.SemaphoreType.DMA,
            pltpu.SemaphoreType.DMA,
            pltpu.SemaphoreType.DMA,
        ],
    )
    def gk(table_hbm, idx_hbm, out_hbm, idx_v, rows_v,
           si0, si1, sg0, sg1, so0, so1):
        wid = lax.axis_index("s") * NC + lax.axis_index("c")
        base = wid * epw
        sis = (si0, si1)
        sgs = (sg0, sg1)
        sos = (so0, so1)

        def fetch_idx(i, bb):
            off = base + i * SUPG
            for j in range(KG):
                pltpu.async_copy(idx_hbm.at[pl.ds(off + j * CH, CH)],
                                 idx_v.at[bb].at[j], sis[bb])

        def drain_idx(i, bb):
            off = base + i * SUPG
            for j in range(KG):
                pltpu.make_async_copy(idx_hbm.at[pl.ds(off + j * CH, CH)],
                                      idx_v.at[bb].at[j], sis[bb]).wait()

        fetch_idx(0, 0)

        def body(i, carry):
            b = lax.rem(i, 2)
            off = base + i * SUPG

            def stage(bb):
                ib = idx_v.at[bb]
                rb = rows_v.at[bb]
                drain_idx(i, bb)

                @pl.when(i + 1 < iters)
                def _():
                    fetch_idx(i + 1, 1 - bb)

                @pl.when(i >= 2)
                def _():
                    pltpu.make_async_copy(
                        rb, out_hbm.at[pl.ds(off - 2 * SUPG, SUPG)],
                        sos[bb]).wait()
                hs = [pltpu.async_copy(
                          table_hbm.at[ib.at[j]],
                          rb.at[pl.ds(j * CH, CH)], sgs[bb])
                      for j in range(KG)]
                for h in hs:
                    h.wait()
                pltpu.async_copy(rb, out_hbm.at[pl.ds(off, SUPG)], sos[bb])

            @pl.when(b == 0)
            def _():
                stage(0)

            @pl.when(b == 1)
            def _():
                stage(1)

            return carry

        lax.fori_loop(0, iters, body, 0)
        for j in (iters - 2, iters - 1):
            pltpu.make_async_copy(
                rows_v.at[j % 2],
                out_hbm.at[pl.ds(base + j * SUPG, SUPG)],
                sos[j % 2]).wait()

    return gk(node_feat, src)


def _sc_scatter(msg, dst, ne):
    mesh = plsc.VectorSubcoreMesh(core_axis_name="c", subcore_axis_name="s")
    zeros = jnp.zeros((RPS, D), jnp.float32)
    epw = ne // NW
    iters = epw // SUPS

    @functools.partial(
        pl.kernel, mesh=mesh,
        out_type=jax.ShapeDtypeStruct((NC * NP, D), jnp.float32),
        scratch_types=[
            pltpu.VMEM((2, KS, CH), jnp.int32),
            pltpu.VMEM((2, SUPS, D), jnp.float32),
            pltpu.VMEM_SHARED((NP, D), jnp.float32),
            pltpu.SemaphoreType.DMA,
            pltpu.SemaphoreType.DMA,
            pltpu.SemaphoreType.DMA,
            pltpu.SemaphoreType.DMA,
        ],
    )
    def sk(msg_hbm, dst_hbm, zero_hbm, out_hbm, idx_v, rows_v,
           acc_sh, si0, si1, sm0, sm1):
        cid = lax.axis_index("c")
        sid = lax.axis_index("s")
        wid = sid * NC + cid
        pltpu.sync_copy(zero_hbm, acc_sh.at[pl.ds(sid * RPS, RPS)])
        plsc.subcore_barrier()

        base = wid * epw
        sis = (si0, si1)
        sms = (sm0, sm1)

        def fetch(i, bb):
            off = base + i * SUPS
            for j in range(KS):
                pltpu.async_copy(dst_hbm.at[pl.ds(off + j * CH, CH)],
                                 idx_v.at[bb].at[j], sis[bb])
            pltpu.async_copy(msg_hbm.at[pl.ds(off, SUPS)],
                             rows_v.at[bb], sms[bb])

        def drain(i, bb):
            off = base + i * SUPS
            for j in range(KS):
                pltpu.make_async_copy(dst_hbm.at[pl.ds(off + j * CH, CH)],
                                      idx_v.at[bb].at[j], sis[bb]).wait()
            pltpu.make_async_copy(msg_hbm.at[pl.ds(off, SUPS)],
                                  rows_v.at[bb], sms[bb]).wait()

        fetch(0, 0)

        def body(i, carry):
            b = lax.rem(i, 2)

            def stage(bb):
                ib = idx_v.at[bb]
                rb = rows_v.at[bb]
                drain(i, bb)

                @pl.when(i + 1 < iters)
                def _():
                    fetch(i + 1, 1 - bb)

                for j in range(KS):
                    pltpu.sync_copy(rb.at[pl.ds(j * CH, CH)],
                                    acc_sh.at[ib.at[j]], add=True)

            @pl.when(b == 0)
            def _():
                stage(0)

            @pl.when(b == 1)
            def _():
                stage(1)

            return carry

        lax.fori_loop(0, iters, body, 0)
        plsc.subcore_barrier()
        pltpu.sync_copy(acc_sh.at[pl.ds(sid * RPS, RPS)],
                        out_hbm.at[pl.ds(cid * NP + sid * RPS, RPS)])

    return sk(msg, dst, zeros)


def _silu(x):
    return x * (1.0 / (1.0 + jnp.exp(-x))) * SILU_NORM


def _tc_msg_body(evt_ref, ev_ref, xg_ref, w0t_ref, w1t_ref, w2a_ref,
                 w2b_ref, t_ref, s_ref, r_ref, out_ref):
    f32 = jnp.float32
    bf16 = jnp.bfloat16
    dn0 = (((0,), (0,)), ((), ()))
    evt = evt_ref[...]
    d2t = (evt[0:1] * evt[0:1] + evt[1:2] * evt[1:2]
           + evt[2:3] * evt[2:3])
    rdt = lax.rsqrt(d2t)
    dt = d2t * rdt
    h = _silu(w0t_ref[...] * dt)
    h = _silu(lax.dot_general(w1t_ref[...], h.astype(bf16),
                              (((1,), (0,)), ((), ())),
                              preferred_element_type=f32) * _ISQH)
    hs = (h * rdt).astype(bf16)
    h = h.astype(bf16)
    ab = lax.dot_general(h, w2a_ref[...], dn0,
                         preferred_element_type=f32) * _ISQH
    t13 = lax.dot_general(hs, w2b_ref[...], dn0,
                          preferred_element_type=f32) * _ISQH
    tile_ev = jnp.dot(ev_ref[...], t_ref[...],
                      preferred_element_type=f32) * jnp.sqrt(3.0)
    x = xg_ref[...]
    x0 = x[:, :MUL]
    x1 = x[:, MUL:]
    s = jnp.dot((x1 * tile_ev).astype(bf16), s_ref[...],
                preferred_element_type=f32)
    r1 = jnp.dot((t13[:, :MUL] * x0).astype(bf16), r_ref[...],
                 preferred_element_type=f32)
    out_ref[:, :MUL] = _C * (ab[:, :MUL] * x0 + t13[:, MUL:] * s)
    out_ref[:, MUL:] = _C * (r1 * tile_ev + ab[:, MUL:] * x1)


_REP3 = np.repeat(np.arange(MUL), 3)


def _tc_msg(evt, evr, xg, fc_w0, fc_w1, fc_w2, ne):
    bs = 2560
    grid = ne // bs
    bf16 = jnp.bfloat16
    w0t = fc_w0.T
    w1t = fc_w1.T.astype(bf16)
    w2a = jnp.concatenate(
        [fc_w2[:, :MUL], fc_w2[:, 2 * MUL:3 * MUL][:, _REP3]],
        1).astype(bf16)
    w2b = jnp.concatenate(
        [fc_w2[:, MUL:2 * MUL], fc_w2[:, 3 * MUL:] * _ISQ3], 1).astype(bf16)
    full = lambda shape: pl.BlockSpec(shape, lambda i: (0, 0))
    return pl.pallas_call(
        _tc_msg_body,
        grid=(grid,),
        in_specs=[
            pl.BlockSpec((3, bs), lambda i: (0, i)),
            pl.BlockSpec((bs, 3), lambda i: (i, 0)),
            pl.BlockSpec((bs, D), lambda i: (i, 0)),
            full((32, 1)), full((32, 32)), full((32, 128)), full((32, 64)),
            full((3, 96)), full((96, 32)), full((32, 96)),
        ],
        out_specs=pl.BlockSpec((bs, D), lambda i: (i, 0)),
        out_shape=jax.ShapeDtypeStruct((ne, D), jnp.float32),
    )(evt, evr, xg, w0t, w1t, w2a, w2b, jnp.asarray(_T),
      jnp.asarray(_S, bf16), jnp.asarray(_R, bf16))


def _tc_final_body(*refs):
    nparts = (len(refs) - 7) // 2
    part_refs = refs[:2 * nparts]
    nf_ref, w0_ref, w1_ref, s_ref, r_ref, k_ref, out_ref = refs[2 * nparts:]
    f32 = jnp.float32
    a = part_refs[0][...]
    for p in part_refs[1:]:
        a = a + p[...]
    s0 = a[:, :MUL]
    s1 = a[:, MUL:]
    y0 = jnp.dot(s0, w0_ref[...], preferred_element_type=f32) * _ISQH
    w1k = jnp.dot(jnp.dot(s_ref[...], w1_ref[...],
                          preferred_element_type=f32),
                  r_ref[...], preferred_element_type=f32) * k_ref[...]
    y1 = jnp.dot(s1, w1k, preferred_element_type=f32) * _ISQH
    out_ref[...] = jnp.concatenate([y0, y1], axis=1) + nf_ref[...]


def _tc_final(parts, node_feat, lin_w0, lin_w1):
    nb = 1280
    grid = (N + nb - 1) // nb
    hi = NP // nb
    full = lambda shape: pl.BlockSpec(shape, lambda i: (0, 0))
    part_specs = []
    part_args = []
    for p in parts:
        part_specs.append(pl.BlockSpec((nb, D), lambda i: (i, 0)))
        part_specs.append(pl.BlockSpec((nb, D), lambda i, h=hi: (i + h, 0)))
        part_args.extend([p, p])
    return pl.pallas_call(
        _tc_final_body,
        grid=(grid,),
        in_specs=part_specs + [
            pl.BlockSpec((nb, D), lambda i: (i, 0)),
            full((32, 32)), full((32, 32)),
            full((96, 32)), full((32, 96)), full((96, 96)),
        ],
        out_specs=pl.BlockSpec((nb, D), lambda i: (i, 0)),
        out_shape=jax.ShapeDtypeStruct((N, D), jnp.float32),
    )(*part_args, node_feat, lin_w0, lin_w1,
      jnp.asarray(_S), jnp.asarray(_R), jnp.asarray(_K))


def kernel(node_feat, edge_index, edge_vec, fc_w0, fc_w1, fc_w2,
           lin_w0, lin_w1):
    src = edge_index[0]
    dst = edge_index[1]
    evt = edge_vec.T
    parts = []
    for c in range(NCHUNK):
        sl = slice(c * EC, (c + 1) * EC)
        xg = _sc_gather(node_feat, src[sl], EC)
        msg = _tc_msg(evt[:, sl], edge_vec[sl], xg, fc_w0, fc_w1, fc_w2, EC)
        parts.append(_sc_scatter(msg, dst[sl], EC))
    return _tc_final(parts, node_feat, lin_w0, lin_w1)


# speedup vs baseline: 1.0368x; 1.0368x over previous
import functools
import numpy as np
import jax
import jax.numpy as jnp
from jax import lax
from jax.experimental import pallas as pl
from jax.experimental.pallas import tpu as pltpu
from jax.experimental.pallas import tpu_sc as plsc

N = 10000
E = 640000
MUL = 32
D = 4 * MUL
SILU_NORM = 1.679
_C = 1.0 / np.sqrt(2.0)
_ISQ3 = 1.0 / np.sqrt(3.0)
_ISQH = 1.0 / np.sqrt(32.0)

NC = 2
NS = 16
NW = NC * NS
CH = 80
NP = 10240
RPS = NP // NS
NCHUNK = 5
EC = E // NCHUNK

_S = np.zeros((96, 32), np.float32)
_R = np.zeros((32, 96), np.float32)
_T = np.zeros((3, 96), np.float32)
for _u in range(32):
    for _m in range(3):
        _S[3 * _u + _m, _u] = 1.0
        _R[_u, 3 * _u + _m] = 1.0
        _T[_m, 3 * _u + _m] = 1.0
_K = (np.arange(96)[:, None] % 3 == np.arange(96)[None, :] % 3)
_K = _K.astype(np.float32)

_BD = np.zeros((160, 224), np.float32)
_BD[0:96, 0:32] = _S
_BD[96:128, 32:128] = _R
_BD[128:160, 128:224] = _R


KG = 5
SUPG = KG * CH
KS = 2
SUPS = KS * CH


def _sc_gather(node_feat, src, ne):
    mesh = plsc.VectorSubcoreMesh(core_axis_name="c", subcore_axis_name="s")
    epw = ne // NW
    iters = epw // SUPG

    @functools.partial(
        pl.kernel, mesh=mesh,
        out_type=jax.ShapeDtypeStruct((ne, D), jnp.float32),
        scratch_types=[
            pltpu.VMEM((2, KG, CH), jnp.int32),
            pltpu.VMEM((2, SUPG, D), jnp.float32),
            pltpu.SemaphoreType.DMA,
            pltpu.SemaphoreType.DMA,
            pltpu.SemaphoreType.DMA,
            pltpu.SemaphoreType.DMA,
            pltpu.SemaphoreType.DMA,
            pltpu.SemaphoreType.DMA,
        ],
    )
    def gk(table_hbm, idx_hbm, out_hbm, idx_v, rows_v,
           si0, si1, sg0, sg1, so0, so1):
        wid = lax.axis_index("s") * NC + lax.axis_index("c")
        base = wid * epw
        sis = (si0, si1)
        sgs = (sg0, sg1)
        sos = (so0, so1)

        def fetch_idx(i, bb):
            off = base + i * SUPG
            for j in range(KG):
                pltpu.async_copy(idx_hbm.at[pl.ds(off + j * CH, CH)],
                                 idx_v.at[bb].at[j], sis[bb])

        def drain_idx(i, bb):
            off = base + i * SUPG
            for j in range(KG):
                pltpu.make_async_copy(idx_hbm.at[pl.ds(off + j * CH, CH)],
                                      idx_v.at[bb].at[j], sis[bb]).wait()

        fetch_idx(0, 0)

        def body(i, carry):
            b = lax.rem(i, 2)
            off = base + i * SUPG

            def stage(bb):
                ib = idx_v.at[bb]
                rb = rows_v.at[bb]
                drain_idx(i, bb)

                @pl.when(i + 1 < iters)
                def _():
                    fetch_idx(i + 1, 1 - bb)

                @pl.when(i >= 2)
                def _():
                    pltpu.make_async_copy(
                        rb, out_hbm.at[pl.ds(off - 2 * SUPG, SUPG)],
                        sos[bb]).wait()
                hs = [pltpu.async_copy(
                          table_hbm.at[ib.at[j]],
                          rb.at[pl.ds(j * CH, CH)], sgs[bb])
                      for j in range(KG)]
                for h in hs:
                    h.wait()
                pltpu.async_copy(rb, out_hbm.at[pl.ds(off, SUPG)], sos[bb])

            @pl.when(b == 0)
            def _():
                stage(0)

            @pl.when(b == 1)
            def _():
                stage(1)

            return carry

        lax.fori_loop(0, iters, body, 0)
        for j in (iters - 2, iters - 1):
            pltpu.make_async_copy(
                rows_v.at[j % 2],
                out_hbm.at[pl.ds(base + j * SUPG, SUPG)],
                sos[j % 2]).wait()

    return gk(node_feat, src)


def _sc_scatter(msg, dst, ne):
    mesh = plsc.VectorSubcoreMesh(core_axis_name="c", subcore_axis_name="s")
    zeros = jnp.zeros((RPS, D), jnp.float32)
    epw = ne // NW
    iters = epw // SUPS

    @functools.partial(
        pl.kernel, mesh=mesh,
        out_type=jax.ShapeDtypeStruct((NC * NP, D), jnp.float32),
        scratch_types=[
            pltpu.VMEM((2, KS, CH), jnp.int32),
            pltpu.VMEM((2, SUPS, D), jnp.float32),
            pltpu.VMEM_SHARED((NP, D), jnp.float32),
            pltpu.SemaphoreType.DMA,
            pltpu.SemaphoreType.DMA,
            pltpu.SemaphoreType.DMA,
            pltpu.SemaphoreType.DMA,
        ],
    )
    def sk(msg_hbm, dst_hbm, zero_hbm, out_hbm, idx_v, rows_v,
           acc_sh, si0, si1, sm0, sm1):
        cid = lax.axis_index("c")
        sid = lax.axis_index("s")
        wid = sid * NC + cid
        pltpu.sync_copy(zero_hbm, acc_sh.at[pl.ds(sid * RPS, RPS)])
        plsc.subcore_barrier()

        base = wid * epw
        sis = (si0, si1)
        sms = (sm0, sm1)

        def fetch(i, bb):
            off = base + i * SUPS
            for j in range(KS):
                pltpu.async_copy(dst_hbm.at[pl.ds(off + j * CH, CH)],
                                 idx_v.at[bb].at[j], sis[bb])
            pltpu.async_copy(msg_hbm.at[pl.ds(off, SUPS)],
                             rows_v.at[bb], sms[bb])

        def drain(i, bb):
            off = base + i * SUPS
            for j in range(KS):
                pltpu.make_async_copy(dst_hbm.at[pl.ds(off + j * CH, CH)],
                                      idx_v.at[bb].at[j], sis[bb]).wait()
            pltpu.make_async_copy(msg_hbm.at[pl.ds(off, SUPS)],
                                  rows_v.at[bb], sms[bb]).wait()

        fetch(0, 0)

        def body(i, carry):
            b = lax.rem(i, 2)

            def stage(bb):
                ib = idx_v.at[bb]
                rb = rows_v.at[bb]
                drain(i, bb)

                @pl.when(i + 1 < iters)
                def _():
                    fetch(i + 1, 1 - bb)

                for j in range(KS):
                    pltpu.sync_copy(rb.at[pl.ds(j * CH, CH)],
                                    acc_sh.at[ib.at[j]], add=True)

            @pl.when(b == 0)
            def _():
                stage(0)

            @pl.when(b == 1)
            def _():
                stage(1)

            return carry

        lax.fori_loop(0, iters, body, 0)
        plsc.subcore_barrier()
        pltpu.sync_copy(acc_sh.at[pl.ds(sid * RPS, RPS)],
                        out_hbm.at[pl.ds(cid * NP + sid * RPS, RPS)])

    return sk(msg, dst, zeros)


def _silu(x):
    return x * (1.0 / (1.0 + jnp.exp(-x))) * SILU_NORM


def _tc_msg_body(evt_ref, xg_ref, w0t_ref, w1t_ref, w2a_ref, w2b_ref,
                 t_ref, bd_ref, out_ref):
    f32 = jnp.float32
    dn0 = (((0,), (0,)), ((), ()))
    evt = evt_ref[...]
    d2t = (evt[0:1] * evt[0:1] + evt[1:2] * evt[1:2]
           + evt[2:3] * evt[2:3])
    bf16 = jnp.bfloat16
    rdt = lax.rsqrt(d2t)
    dt = d2t * rdt
    h = _silu(w0t_ref[...] * dt)
    h = _silu(lax.dot_general(w1t_ref[...], h.astype(bf16),
                              (((1,), (0,)), ((), ())),
                              preferred_element_type=f32) * _ISQH)
    hs = (h * rdt).astype(bf16)
    h = h.astype(bf16)
    tpw02 = lax.dot_general(h, w2a_ref[...], dn0,
                            preferred_element_type=f32) * _ISQH
    tpw13 = lax.dot_general(hs, w2b_ref[...], dn0,
                            preferred_element_type=f32) * _ISQH
    tile_ev = lax.dot_general(evt, t_ref[...], dn0,
                              preferred_element_type=f32) * jnp.sqrt(3.0)
    x = xg_ref[...].astype(f32)
    x0 = x[:, :MUL]
    x1 = x[:, MUL:]
    a = jnp.concatenate(
        [x1 * tile_ev, tpw13[:, :MUL] * x0, tpw02[:, MUL:]],
        axis=1).astype(bf16)
    srr = jnp.dot(a, bd_ref[...], preferred_element_type=f32)
    out_ref[:, :MUL] = _C * (tpw02[:, :MUL] * x0
                             + tpw13[:, MUL:] * srr[:, :MUL])
    out_ref[:, MUL:] = _C * (srr[:, MUL:MUL + 96] * tile_ev
                             + srr[:, MUL + 96:] * x1)


def _tc_msg(evt, xg, fc_w0, fc_w1, fc_w2, ne):
    bs = 2560
    grid = ne // bs
    bf16 = jnp.bfloat16
    w0t = fc_w0.T
    w1t = fc_w1.T.astype(bf16)
    w2a = jnp.concatenate(
        [fc_w2[:, :MUL], fc_w2[:, 2 * MUL:3 * MUL]], 1).astype(bf16)
    w2b = jnp.concatenate(
        [fc_w2[:, MUL:2 * MUL], fc_w2[:, 3 * MUL:] * _ISQ3], 1).astype(bf16)
    full = lambda shape: pl.BlockSpec(shape, lambda i: (0, 0))
    return pl.pallas_call(
        _tc_msg_body,
        grid=(grid,),
        in_specs=[
            pl.BlockSpec((3, bs), lambda i: (0, i)),
            pl.BlockSpec((bs, D), lambda i: (i, 0)),
            full((32, 1)), full((32, 32)), full((32, 64)), full((32, 64)),
            full((3, 96)), full((160, 224)),
        ],
        out_specs=pl.BlockSpec((bs, D), lambda i: (i, 0)),
        out_shape=jax.ShapeDtypeStruct((ne, D), jnp.float32),
    )(evt, xg, w0t, w1t, w2a, w2b, jnp.asarray(_T),
      jnp.asarray(_BD, bf16))


def _tc_final_body(*refs):
    nparts = (len(refs) - 7) // 2
    part_refs = refs[:2 * nparts]
    nf_ref, w0_ref, w1_ref, s_ref, r_ref, k_ref, out_ref = refs[2 * nparts:]
    f32 = jnp.float32
    a = part_refs[0][...]
    for p in part_refs[1:]:
        a = a + p[...]
    s0 = a[:, :MUL]
    s1 = a[:, MUL:]
    y0 = jnp.dot(s0, w0_ref[...], preferred_element_type=f32) * _ISQH
    w1k = jnp.dot(jnp.dot(s_ref[...], w1_ref[...],
                          preferred_element_type=f32),
                  r_ref[...], preferred_element_type=f32) * k_ref[...]
    y1 = jnp.dot(s1, w1k, preferred_element_type=f32) * _ISQH
    out_ref[...] = jnp.concatenate([y0, y1], axis=1) + nf_ref[...]


def _tc_final(parts, node_feat, lin_w0, lin_w1):
    nb = 1280
    grid = (N + nb - 1) // nb
    hi = NP // nb
    full = lambda shape: pl.BlockSpec(shape, lambda i: (0, 0))
    part_specs = []
    part_args = []
    for p in parts:
        part_specs.append(pl.BlockSpec((nb, D), lambda i: (i, 0)))
        part_specs.append(pl.BlockSpec((nb, D), lambda i, h=hi: (i + h, 0)))
        part_args.extend([p, p])
    return pl.pallas_call(
        _tc_final_body,
        grid=(grid,),
        in_specs=part_specs + [
            pl.BlockSpec((nb, D), lambda i: (i, 0)),
            full((32, 32)), full((32, 32)),
            full((96, 32)), full((32, 96)), full((96, 96)),
        ],
        out_specs=pl.BlockSpec((nb, D), lambda i: (i, 0)),
        out_shape=jax.ShapeDtypeStruct((N, D), jnp.float32),
    )(*part_args, node_feat, lin_w0, lin_w1,
      jnp.asarray(_S), jnp.asarray(_R), jnp.asarray(_K))


def kernel(node_feat, edge_index, edge_vec, fc_w0, fc_w1, fc_w2,
           lin_w0, lin_w1):
    src = edge_index[0]
    dst = edge_index[1]
    evt = edge_vec.T
    parts = []
    for c in range(NCHUNK):
        sl = slice(c * EC, (c + 1) * EC)
        xg = _sc_gather(node_feat, src[sl], EC)
        msg = _tc_msg(evt[:, sl], xg, fc_w0, fc_w1, fc_w2, EC)
        parts.append(_sc_scatter(msg, dst[sl], EC))
    return _tc_final(parts, node_feat, lin_w0, lin_w1)
